# baseline (speedup 1.0000x reference)
import functools

import jax
import jax.numpy as jnp
from jax import lax
from jax.experimental import pallas as pl
from jax.experimental.pallas import tpu as pltpu
from jax.experimental.pallas import tpu_sc as plsc

NUM_NODES = 100000
NUM_CELLS = 200000
CPAD = 200000
FEAT = 128
PRE_W = 9
ROW_W = PRE_W + 3 * FEAT
NWORKERS = 32
NJ = (ROW_W + NWORKERS - 1) // NWORKERS
CH = 2000
NCH = NUM_CELLS // CH
NBUF = 4


def _sc_assemble_cols(ut_flat, trit_flat, pre_flat):
    mesh = plsc.VectorSubcoreMesh(core_axis_name="c", subcore_axis_name="s")

    @functools.partial(
        pl.kernel,
        mesh=mesh,
        out_type=jax.ShapeDtypeStruct((ROW_W * CPAD,), jnp.float32),
        scratch_types=[
            pltpu.VMEM((NUM_NODES,), jnp.float32),
            pltpu.VMEM((CH,), jnp.int32),
            pltpu.VMEM((CH,), jnp.int32),
            pltpu.VMEM((CH,), jnp.int32),
            pltpu.VMEM((CH,), jnp.int32),
            pltpu.VMEM((CH,), jnp.float32),
            pltpu.VMEM((CH,), jnp.float32),
            pltpu.VMEM((CH,), jnp.float32),
            pltpu.VMEM((CH,), jnp.float32),
            pltpu.SemaphoreType.DMA,
            pltpu.SemaphoreType.DMA,
        ],
        compiler_params=pltpu.CompilerParams(needs_layout_passes=False),
    )
    def asm(
        ut_hbm, trit_hbm, pre_hbm, out_hbm,
        urow_v, idx0_v, idx1_v, idx2_v, idx3_v,
        out0_v, out1_v, out2_v, out3_v, idx_sem, out_sem,
    ):
        wid = lax.axis_index("s") * 2 + lax.axis_index("c")
        idx_bufs = (idx0_v, idx1_v, idx2_v, idx3_v)
        out_bufs = (out0_v, out1_v, out2_v, out3_v)

        def idx_copy(v, c0, b):
            return pltpu.make_async_copy(
                trit_hbm.at[pl.ds(v * NUM_CELLS + c0, CH)], idx_bufs[b], idx_sem
            )

        def out_copy(j, c0, b):
            return pltpu.make_async_copy(
                out_bufs[b], out_hbm.at[pl.ds(j * CPAD + c0, CH)], out_sem
            )

        def gather_chunk(b):
            for i in range(CH // 16):
                vec = idx_bufs[b][pl.ds(i * 16, 16)]
                out_bufs[b][pl.ds(i * 16, 16)] = plsc.load_gather(urow_v, [vec])

        def col_body(jj, carry):
            j = wid + NWORKERS * jj

            @pl.when(j < PRE_W)
            def _():
                def pre_read(c0, b):
                    return pltpu.make_async_copy(
                        pre_hbm.at[pl.ds(j * NUM_CELLS + c0, CH)],
                        out_bufs[b],
                        idx_sem,
                    )

                for b in range(NBUF):
                    pre_read(b * CH, b).start()

                def pchunk(kkq, carry2):
                    k0 = NBUF * kkq
                    for b in range(NBUF):
                        pre_read((k0 + b) * CH, b).wait()
                        out_copy(j, (k0 + b) * CH, b).start()

                    @pl.when(kkq < NCH // NBUF - 1)
                    def _():
                        for b in range(NBUF):
                            out_copy(j, 0, b).wait()
                            pre_read((k0 + NBUF + b) * CH, b).start()

                    return carry2

                lax.fori_loop(0, NCH // NBUF, pchunk, 0)
                for b in range(NBUF):
                    out_copy(j, 0, b).wait()

            @pl.when((j >= PRE_W) & (j < ROW_W))
            def _():
                g = j - PRE_W
                v = g // FEAT
                f = g % FEAT
                pltpu.sync_copy(
                    ut_hbm.at[pl.ds(f * NUM_NODES, NUM_NODES)], urow_v
                )
                for b in range(NBUF):
                    idx_copy(v, b * CH, b).start()

                def pipe(kkq, carry2):
                    k0 = NBUF * kkq
                    for b in range(NBUF):
                        idx_copy(v, (k0 + b) * CH, b).wait()

                        @pl.when(kkq > 0)
                        def _():
                            out_copy(j, 0, b).wait()

                        gather_chunk(b)
                        out_copy(j, (k0 + b) * CH, b).start()

                        @pl.when(kkq < NCH // NBUF - 1)
                        def _():
                            idx_copy(v, (k0 + NBUF + b) * CH, b).start()

                    return carry2

                lax.fori_loop(0, NCH // NBUF, pipe, 0)
                for b in range(NBUF):
                    out_copy(j, 0, b).wait()

            return carry

        lax.fori_loop(0, NJ, col_body, 0)

    return asm(ut_flat, trit_flat, pre_flat)


def kernel(u, t, cell_centers, cell_local_vertex_pos, triangulation):
    n_nodes, feat = u.shape[1], u.shape[2]
    ncells = triangulation.shape[0]
    ut_flat = jnp.transpose(u.reshape(n_nodes, feat)).reshape(n_nodes * feat)
    trit_flat = (
        jnp.transpose(triangulation.astype(jnp.int32)).reshape(3 * ncells)
    )
    vp = cell_local_vertex_pos.reshape(ncells, 6)
    pre_flat = jnp.concatenate(
        [jnp.broadcast_to(t.reshape(1, 1), (ncells, 1)), cell_centers, vp],
        axis=1,
    )
    pre_flat = jnp.transpose(pre_flat).reshape(PRE_W * ncells)
    full = _sc_assemble_cols(ut_flat, trit_flat, pre_flat)
    out = jnp.transpose(full.reshape(ROW_W, CPAD))
    return out[None]
